# baseline (speedup 1.0000x reference)
import functools

import jax
import jax.numpy as jnp
from jax import lax
from jax.experimental import pallas as pl
from jax.experimental.pallas import tpu as pltpu
from jax.experimental.pallas import tpu_sc as plsc

_LANES = 16
_IDX_W = 128
_WAVE = 8


def _build_sc_call(B, F, V):
    NW = 32
    BPW = B // NW
    IPW = BPW * F
    RPW = IPW // _IDX_W
    RCH = BPW // _LANES

    mesh = plsc.VectorSubcoreMesh(core_axis_name="c", subcore_axis_name="s")

    @functools.partial(
        pl.kernel,
        out_type=jax.ShapeDtypeStruct((B,), jnp.float32),
        mesh=mesh,
        compiler_params=pltpu.CompilerParams(needs_layout_passes=False),
        scratch_types=[
            pltpu.VMEM((RPW, _IDX_W), jnp.int32),
            pltpu.VMEM((IPW,), jnp.float32),
            pltpu.VMEM((_LANES,), jnp.float32),
            pltpu.VMEM((BPW,), jnp.float32),
            pltpu.SemaphoreType.DMA,
        ],
    )
    def sc_call(x_hbm, fc_hbm, bias_hbm, out_hbm, idx_v, vals_v, bias_v,
                out_v, sem):
        wid = lax.axis_index("s") * 2 + lax.axis_index("c")

        pltpu.sync_copy(x_hbm.at[pl.ds(wid * RPW, RPW)], idx_v)
        pltpu.sync_copy(bias_hbm, bias_v)

        @pl.loop(0, RPW // _WAVE)
        def _gather_wave(w):
            handles = []
            for b in range(_WAVE):
                j = w * _WAVE + b
                handles.append(
                    pltpu.async_copy(fc_hbm.at[idx_v.at[j]],
                                     vals_v.at[pl.ds(j * _IDX_W, _IDX_W)],
                                     sem))
            for h in handles:
                h.wait()

        @pl.loop(0, RCH)
        def _reduce(c):
            iot = lax.iota(jnp.int32, _LANES)
            q0 = (c * _LANES + iot) * F
            acc = bias_v[...]
            for f in range(F):
                acc = acc + plsc.load_gather(vals_v, [q0 + f])
            out_v[pl.ds(c * _LANES, _LANES)] = acc

        pltpu.sync_copy(out_v, out_hbm.at[pl.ds(wid * BPW, BPW)])

    return sc_call


def kernel(x, fc, bias):
    B, F = x.shape
    V, OD = fc.shape
    xf = x.astype(jnp.int32).reshape((B * F) // _IDX_W, _IDX_W)
    fcf = fc.reshape(V)
    bias16 = jnp.broadcast_to(bias.astype(jnp.float32), (_LANES,))
    out = _build_sc_call(B, F, V)(xf, fcf, bias16)
    return out.reshape(B, OD)


# speedup vs baseline: 2.4928x; 2.4928x over previous
import functools

import jax
import jax.numpy as jnp
from jax import lax
from jax.experimental import pallas as pl
from jax.experimental.pallas import tpu as pltpu
from jax.experimental.pallas import tpu_sc as plsc

_LANES = 16
_IDX_W = 128


def _build_sc_call(B, F, VP):
    NW = 32
    BPW = B // NW
    KW = BPW // _IDX_W
    RCH = BPW // _LANES

    mesh = plsc.VectorSubcoreMesh(core_axis_name="c", subcore_axis_name="s")

    @functools.partial(
        pl.kernel,
        out_type=jax.ShapeDtypeStruct((B,), jnp.float32),
        mesh=mesh,
        scratch_types=[
            pltpu.VMEM((F, BPW), jnp.int32),
            pltpu.VMEM((F, BPW), jnp.float32),
            pltpu.VMEM((_LANES,), jnp.float32),
            pltpu.VMEM((BPW,), jnp.float32),
            pltpu.SemaphoreType.DMA,
        ],
    )
    def sc_call(xt_hbm, fc_hbm, bias_hbm, out_hbm, idx_v, vals_v, bias_v,
                out_v, sem):
        wid = lax.axis_index("s") * 2 + lax.axis_index("c")
        base = wid * BPW

        pltpu.sync_copy(xt_hbm.at[:, pl.ds(base, BPW)], idx_v)
        pltpu.sync_copy(bias_hbm, bias_v)

        @pl.loop(0, F)
        def _gather_row(f):
            handles = []
            for k in range(KW):
                sl = pl.ds(k * _IDX_W, _IDX_W)
                handles.append(
                    pltpu.async_copy(fc_hbm.at[idx_v.at[f, sl]],
                                     vals_v.at[f, sl], sem))
            for h in handles:
                h.wait()

        @pl.loop(0, RCH)
        def _reduce(c):
            sl = pl.ds(c * _LANES, _LANES)
            acc = bias_v[...]
            for f in range(F):
                acc = acc + vals_v[f, sl]
            out_v[sl] = acc

        pltpu.sync_copy(out_v, out_hbm.at[pl.ds(base, BPW)])

    return sc_call


def kernel(x, fc, bias):
    B, F = x.shape
    V, OD = fc.shape
    xt = x.astype(jnp.int32).T
    vp = (V + 8 * _IDX_W - 1) // (8 * _IDX_W) * (8 * _IDX_W)
    fcf = jnp.pad(fc, ((0, vp - V), (0, 0))).reshape(vp)
    bias16 = jnp.broadcast_to(bias.astype(jnp.float32), (_LANES,))
    out = _build_sc_call(B, F, vp)(xt, fcf, bias16)
    return out.reshape(B, OD)


# speedup vs baseline: 2.7922x; 1.1201x over previous
import functools

import jax
import jax.numpy as jnp
from jax import lax
from jax.experimental import pallas as pl
from jax.experimental.pallas import tpu as pltpu
from jax.experimental.pallas import tpu_sc as plsc

_LANES = 16
_IDX_W = 128


def _build_sc_call(B, F, VP):
    NW = 32
    BPW = B // NW
    KW = BPW // _IDX_W
    RCH = BPW // _LANES

    mesh = plsc.VectorSubcoreMesh(core_axis_name="c", subcore_axis_name="s")

    @functools.partial(
        pl.kernel,
        out_type=jax.ShapeDtypeStruct((B,), jnp.float32),
        mesh=mesh,
        scratch_types=[
            pltpu.VMEM((F, BPW), jnp.int32),
            pltpu.VMEM((F, BPW), jnp.float32),
            pltpu.VMEM((_LANES,), jnp.float32),
            pltpu.VMEM((BPW,), jnp.float32),
            pltpu.SemaphoreType.DMA,
        ],
    )
    def sc_call(xt_hbm, fc_hbm, bias_hbm, out_hbm, idx_v, vals_v, bias_v,
                out_v, sem):
        wid = lax.axis_index("s") * 2 + lax.axis_index("c")
        base = wid * BPW

        pltpu.sync_copy(xt_hbm.at[:, pl.ds(base, BPW)], idx_v)
        pltpu.sync_copy(bias_hbm, bias_v)

        def _fire_row(f):
            for k in range(KW):
                sl = pl.ds(k * _IDX_W, _IDX_W)
                pltpu.async_copy(fc_hbm.at[idx_v.at[f, sl]],
                                 vals_v.at[f, sl], sem)

        def _drain_row():
            for k in range(KW):
                sl = pl.ds(k * _IDX_W, _IDX_W)
                pltpu.make_async_copy(fc_hbm.at[idx_v.at[0, sl]],
                                      vals_v.at[0, sl], sem).wait()

        _fire_row(0)

        @pl.loop(1, F)
        def _gather_row(f):
            _fire_row(f)
            _drain_row()

        _drain_row()

        @pl.loop(0, RCH)
        def _reduce(c):
            sl = pl.ds(c * _LANES, _LANES)
            acc = bias_v[...]
            for f in range(F):
                acc = acc + vals_v[f, sl]
            out_v[sl] = acc

        pltpu.sync_copy(out_v, out_hbm.at[pl.ds(base, BPW)])

    return sc_call


def kernel(x, fc, bias):
    B, F = x.shape
    V, OD = fc.shape
    xt = x.astype(jnp.int32).T
    vp = (V + 8 * _IDX_W - 1) // (8 * _IDX_W) * (8 * _IDX_W)
    fcf = jnp.pad(fc, ((0, vp - V), (0, 0))).reshape(vp)
    bias16 = jnp.broadcast_to(bias.astype(jnp.float32), (_LANES,))
    out = _build_sc_call(B, F, vp)(xt, fcf, bias16)
    return out.reshape(B, OD)


# speedup vs baseline: 2.9399x; 1.0529x over previous
import functools

import jax
import jax.numpy as jnp
from jax import lax
from jax.experimental import pallas as pl
from jax.experimental.pallas import tpu as pltpu
from jax.experimental.pallas import tpu_sc as plsc

_LANES = 16
_IDX_W = 128


def _build_sc_call(B, F, VP):
    NW = 32
    BPW = B // NW
    KW = BPW // _IDX_W
    RCH = BPW // _LANES

    mesh = plsc.VectorSubcoreMesh(core_axis_name="c", subcore_axis_name="s")

    @functools.partial(
        pl.kernel,
        out_type=jax.ShapeDtypeStruct((B,), jnp.float32),
        mesh=mesh,
        scratch_types=[
            pltpu.VMEM((F, BPW), jnp.int32),
            pltpu.VMEM((F, BPW), jnp.float32),
            pltpu.VMEM((_LANES,), jnp.float32),
            pltpu.VMEM((BPW,), jnp.float32),
            pltpu.SemaphoreType.DMA,
        ],
    )
    def sc_call(xt_hbm, fc_hbm, bias_hbm, out_hbm, idx_v, vals_v, bias_v,
                out_v, sem):
        wid = lax.axis_index("s") * 2 + lax.axis_index("c")
        base = wid * BPW

        pltpu.sync_copy(xt_hbm.at[:, pl.ds(base, BPW)], idx_v)
        pltpu.sync_copy(bias_hbm, bias_v)

        def _fire_row(f):
            for k in range(KW):
                sl = pl.ds(k * _IDX_W, _IDX_W)
                pltpu.async_copy(fc_hbm.at[idx_v.at[f, sl]],
                                 vals_v.at[f, sl], sem)

        def _drain_row():
            for k in range(KW):
                sl = pl.ds(k * _IDX_W, _IDX_W)
                pltpu.make_async_copy(fc_hbm.at[idx_v.at[0, sl]],
                                      vals_v.at[0, sl], sem).wait()

        DEPTH = 4
        for f in range(DEPTH - 1):
            _fire_row(f)

        @pl.loop(DEPTH - 1, F)
        def _gather_row(f):
            _fire_row(f)
            _drain_row()

        for _ in range(DEPTH - 1):
            _drain_row()

        @pl.loop(0, RCH)
        def _reduce(c):
            sl = pl.ds(c * _LANES, _LANES)
            acc = bias_v[...]
            for f in range(F):
                acc = acc + vals_v[f, sl]
            out_v[sl] = acc

        pltpu.sync_copy(out_v, out_hbm.at[pl.ds(base, BPW)])

    return sc_call


def kernel(x, fc, bias):
    B, F = x.shape
    V, OD = fc.shape
    xt = x.astype(jnp.int32).T
    vp = (V + 8 * _IDX_W - 1) // (8 * _IDX_W) * (8 * _IDX_W)
    fcf = jnp.pad(fc, ((0, vp - V), (0, 0))).reshape(vp)
    bias16 = jnp.broadcast_to(bias.astype(jnp.float32), (_LANES,))
    out = _build_sc_call(B, F, vp)(xt, fcf, bias16)
    return out.reshape(B, OD)


# speedup vs baseline: 3.0359x; 1.0327x over previous
import functools

import jax
import jax.numpy as jnp
from jax import lax
from jax.experimental import pallas as pl
from jax.experimental.pallas import tpu as pltpu
from jax.experimental.pallas import tpu_sc as plsc

_LANES = 16
_IDX_W = 128


def _build_sc_call(B, F, VP):
    NW = 32
    BPW = B // NW
    KW = BPW // _IDX_W
    RCH = BPW // _LANES

    mesh = plsc.VectorSubcoreMesh(core_axis_name="c", subcore_axis_name="s")

    @functools.partial(
        pl.kernel,
        out_type=jax.ShapeDtypeStruct((B,), jnp.float32),
        mesh=mesh,
        scratch_types=[
            pltpu.VMEM((F, BPW), jnp.int32),
            pltpu.VMEM((F, BPW), jnp.float32),
            pltpu.VMEM((_LANES,), jnp.float32),
            pltpu.VMEM((BPW,), jnp.float32),
            pltpu.SemaphoreType.DMA,
        ],
    )
    def sc_call(xt_hbm, fc_hbm, bias_hbm, out_hbm, idx_v, vals_v, bias_v,
                out_v, sem):
        wid = lax.axis_index("s") * 2 + lax.axis_index("c")
        base = wid * BPW

        pltpu.sync_copy(xt_hbm.at[:, pl.ds(base, BPW)], idx_v)
        pltpu.sync_copy(bias_hbm, bias_v)

        def _fire_row(f):
            for k in range(KW):
                sl = pl.ds(k * _IDX_W, _IDX_W)
                pltpu.async_copy(fc_hbm.at[idx_v.at[f, sl]],
                                 vals_v.at[f, sl], sem)

        def _drain_row():
            for k in range(KW):
                sl = pl.ds(k * _IDX_W, _IDX_W)
                pltpu.make_async_copy(fc_hbm.at[idx_v.at[0, sl]],
                                      vals_v.at[0, sl], sem).wait()

        DEPTH = 8
        for f in range(DEPTH - 1):
            _fire_row(f)

        @pl.loop(DEPTH - 1, F)
        def _gather_row(f):
            _fire_row(f)
            _drain_row()

        for _ in range(DEPTH - 1):
            _drain_row()

        @pl.loop(0, RCH)
        def _reduce(c):
            sl = pl.ds(c * _LANES, _LANES)
            acc = bias_v[...]
            for f in range(F):
                acc = acc + vals_v[f, sl]
            out_v[sl] = acc

        pltpu.sync_copy(out_v, out_hbm.at[pl.ds(base, BPW)])

    return sc_call


def kernel(x, fc, bias):
    B, F = x.shape
    V, OD = fc.shape
    xt = x.astype(jnp.int32).T
    vp = (V + 8 * _IDX_W - 1) // (8 * _IDX_W) * (8 * _IDX_W)
    fcf = jnp.pad(fc, ((0, vp - V), (0, 0))).reshape(vp)
    bias16 = jnp.broadcast_to(bias.astype(jnp.float32), (_LANES,))
    out = _build_sc_call(B, F, vp)(xt, fcf, bias16)
    return out.reshape(B, OD)
